# baseline (speedup 1.0000x reference)
import functools
import math

import jax
import jax.numpy as jnp
from jax.experimental import pallas as pl
from jax.experimental.pallas import tpu as pltpu

_BF16 = jnp.bfloat16
_SQRT2 = math.sqrt(2.0)

_CH = 512
_H = 8
_HD = 64
_NQ, _NA, _NC = 512, 1024, 768
_NTOT = _NQ + _NA + _NC
_ROWS = {"question": (0, 512), "answer": (512, 1536), "concept": (1536, 2304)}
_NEG = -1e30


def _erf(x):
    a1, a2, a3, a4, a5 = 0.254829592, -0.284496736, 1.421413741, -1.453152027, 1.061405429
    p = 0.3275911
    sgn = jnp.where(x >= 0.0, 1.0, -1.0)
    ax = jnp.abs(x)
    t = 1.0 / (1.0 + p * ax)
    poly = ((((a5 * t + a4) * t + a3) * t + a2) * t + a1) * t
    return sgn * (1.0 - poly * jnp.exp(-ax * ax))


def _gelu_exact(x):
    return 0.5 * x * (1.0 + _erf(x / _SQRT2))


def _dot(a, b):
    return jnp.dot(a, b, preferred_element_type=jnp.float32)


def _dot_nt(a, b):
    return jax.lax.dot_general(a, b, (((1,), (1,)), ((), ())),
                               preferred_element_type=jnp.float32)


def _attend(hb_in, dst, srcs, qw, qb, alin_w, alin_b, alpha, lc, kc, vc, write):
    d0, d1 = _ROWS[dst]
    hd = hb_in[d0:d1]
    q = (_dot(hd, qw[...]) + qb[...]).astype(_BF16)
    off = 0
    for (s0, s1), kw, kb, vw, vb in srcs:
        hs = hb_in[s0:s1]
        ns = s1 - s0
        kc[off:off + ns] = (_dot(hs, kw[...]) + kb[...]).astype(_BF16)
        vc[off:off + ns] = (_dot(hs, vw[...]) + vb[...]).astype(_BF16)
        off += ns
    spans = []
    o = 0
    for (s0, s1), _, _, _, _ in srcs:
        spans.append((o, o + (s1 - s0)))
        o += s1 - s0
    lcv = lc[...]
    outs = []
    for h in range(_H):
        sl = slice(h * _HD, (h + 1) * _HD)
        t = _dot_nt(q[:, sl], kc[:, sl]) + lcv
        parts = []
        for o0, o1 in spans:
            te = t[:, o0:o1]
            rm = jnp.max(te, axis=-1, keepdims=True)
            ok = rm > -1e29
            w = jnp.exp(te - rm)
            denom = jnp.sum(w, axis=-1, keepdims=True)
            inv = jnp.where(ok, 1.0 / denom, 0.0)
            parts.append(w * inv)
        wn = parts[0] if len(parts) == 1 else jnp.concatenate(parts, axis=-1)
        outs.append(_dot(wn.astype(_BF16), vc[:off, sl]))
    att = jnp.concatenate(outs, axis=-1)
    g = _gelu_exact(att).astype(_BF16)
    y = _dot(g, alin_w[...]) + alin_b[...]
    a = alpha[...]
    write(a * y + (1.0 - a) * hd.astype(jnp.float32))


def _layer(hb_in, lw, lcs, writers):
    specs = [
        ("question", 1792,
         [(_ROWS["answer"],) + lw["k_rev_has"] + lw["v_rev_has"],
          (_ROWS["concept"],) + lw["k_rev_mentions"] + lw["v_rev_mentions"]]),
        ("answer", 512, [(_ROWS["question"],) + lw["k_has"] + lw["v_has"]]),
        ("concept", 512, [(_ROWS["question"],) + lw["k_mentions"] + lw["v_mentions"]]),
    ]
    for dst, ns_tot, srcs in specs:
        fn = functools.partial(
            _attend, hb_in, dst,
            [(rng, kw, kb, vw, vb) for rng, kw, kb, vw, vb in srcs],
            lw["q_" + dst][0], lw["q_" + dst][1],
            lw["alin_" + dst][0], lw["alin_" + dst][1],
            lw["alpha_" + dst], lcs[dst])
        pl.run_scoped(functools.partial(lambda f, w, kc, vc: f(kc, vc, w),
                                        fn, writers[dst]),
                      pltpu.VMEM((ns_tot, _CH), _BF16),
                      pltpu.VMEM((ns_tot, _CH), _BF16))


def _build_lc(e_ref, nd, ns, out_ref, col0):
    ne = e_ref.shape[1]
    dt = jnp.float8_e4m3fn

    def f(a_ref, b_ref):
        a_ref[...] = (jax.lax.broadcasted_iota(jnp.int32, (nd, ne), 0)
                      == e_ref[1:2, :]).astype(dt)
        b_ref[...] = (jax.lax.broadcasted_iota(jnp.int32, (ns, ne), 0)
                      == e_ref[0:1, :]).astype(dt)
        cnt = _dot_nt(a_ref[...], b_ref[...])
        out_ref[:, col0:col0 + ns] = jnp.where(cnt > 0.0, jnp.log(cnt), _NEG)

    pl.run_scoped(f, pltpu.VMEM((nd, ne), dt), pltpu.VMEM((ns, ne), dt))


def _body(xq, xa, xc, w_lin, w_qa, w_kv, rows, b_kv,
          e_has, e_rev_has, e_mentions, e_rev_mentions,
          out_q, out_a, out_c, hb0, hb1, lc_q, lc_a, lc_c):
    xs = {"question": xq, "answer": xa, "concept": xc}
    lin_w = {"question": w_lin.at[0:256], "answer": w_lin.at[256:384],
             "concept": w_lin.at[384:512]}
    types = ("question", "answer", "concept")
    ets = ("has", "rev_has", "mentions", "rev_mentions")
    lin = {t: (lin_w[t], rows.at[i:i + 1], rows.at[3 + i:4 + i],
               rows.at[6 + i:7 + i]) for i, t in enumerate(types)}

    layers = []
    for L in range(2):
        lw = {}
        for i, t in enumerate(types):
            lw["q_" + t] = (w_qa.at[6 * L + i], rows.at[9 + 6 * L + i:10 + 6 * L + i])
            lw["alin_" + t] = (w_qa.at[6 * L + 3 + i],
                               rows.at[12 + 6 * L + i:13 + 6 * L + i])
            lw["alpha_" + t] = rows.at[21 + 3 * L + i:22 + 3 * L + i]
        for j, et in enumerate(ets):
            lw["k_" + et] = (w_kv.at[8 * L + 2 * j], b_kv.at[8 * L + 2 * j:8 * L + 2 * j + 1])
            lw["v_" + et] = (w_kv.at[8 * L + 2 * j + 1], b_kv.at[8 * L + 2 * j + 1:8 * L + 2 * j + 2])
        layers.append(lw)
    _build_lc(e_rev_has, _NQ, _NA, lc_q, 0)
    _build_lc(e_rev_mentions, _NQ, _NC, lc_q, _NA)
    _build_lc(e_has, _NA, _NQ, lc_a, 0)
    _build_lc(e_mentions, _NC, _NQ, lc_c, 0)
    lcs = {"question": lc_q, "answer": lc_a, "concept": lc_c}

    for t in ("question", "answer", "concept"):
        r0, r1 = _ROWS[t]
        w, b, gamma, beta = lin[t]
        y = _dot(xs[t][...], w[...]) + b[...]
        y = jnp.maximum(y, 0.0)
        n = r1 - r0
        mean = jnp.sum(y, axis=0, keepdims=True) * (1.0 / n)
        yc = y - mean
        var = jnp.sum(yc * yc, axis=0, keepdims=True) * (1.0 / n)
        y = yc * jax.lax.rsqrt(var + 1e-5) * gamma[...] + beta[...]
        hb0[r0:r1] = y.astype(_BF16)

    def w0(dst):
        def wr(v):
            r0, r1 = _ROWS[dst]
            hb1[r0:r1] = v.astype(_BF16)
        return wr

    _layer(hb0, layers[0], lcs,
           {d: w0(d) for d in ("question", "answer", "concept")})

    outs = {"question": out_q, "answer": out_a, "concept": out_c}

    def w1(dst):
        def wr(v):
            outs[dst][...] = v
        return wr

    _layer(hb1, layers[1], lcs,
           {d: w1(d) for d in ("question", "answer", "concept")})


def kernel(lin_w_question, lin_b_question, bn_gamma_question, bn_beta_question, lin_w_answer, lin_b_answer, bn_gamma_answer, bn_beta_answer, lin_w_concept, lin_b_concept, bn_gamma_concept, bn_beta_concept, c0_k_w_question, c0_k_b_question, c0_q_w_question, c0_q_b_question, c0_v_w_question, c0_v_b_question, c0_alin_w_question, c0_alin_b_question, c0_skip_question, c0_k_w_answer, c0_k_b_answer, c0_q_w_answer, c0_q_b_answer, c0_v_w_answer, c0_v_b_answer, c0_alin_w_answer, c0_alin_b_answer, c0_skip_answer, c0_k_w_concept, c0_k_b_concept, c0_q_w_concept, c0_q_b_concept, c0_v_w_concept, c0_v_b_concept, c0_alin_w_concept, c0_alin_b_concept, c0_skip_concept, c0_arel_question_has_answer, c0_mrel_question_has_answer, c0_prel_question_has_answer, c0_arel_answer_rev_has_question, c0_mrel_answer_rev_has_question, c0_prel_answer_rev_has_question, c0_arel_question_mentions_concept, c0_mrel_question_mentions_concept, c0_prel_question_mentions_concept, c0_arel_concept_rev_mentions_question, c0_mrel_concept_rev_mentions_question, c0_prel_concept_rev_mentions_question, c1_k_w_question, c1_k_b_question, c1_q_w_question, c1_q_b_question, c1_v_w_question, c1_v_b_question, c1_alin_w_question, c1_alin_b_question, c1_skip_question, c1_k_w_answer, c1_k_b_answer, c1_q_w_answer, c1_q_b_answer, c1_v_w_answer, c1_v_b_answer, c1_alin_w_answer, c1_alin_b_answer, c1_skip_answer, c1_k_w_concept, c1_k_b_concept, c1_q_w_concept, c1_q_b_concept, c1_v_w_concept, c1_v_b_concept, c1_alin_w_concept, c1_alin_b_concept, c1_skip_concept, c1_arel_question_has_answer, c1_mrel_question_has_answer, c1_prel_question_has_answer, c1_arel_answer_rev_has_question, c1_mrel_answer_rev_has_question, c1_prel_answer_rev_has_question, c1_arel_question_mentions_concept, c1_mrel_question_mentions_concept, c1_prel_question_mentions_concept, c1_arel_concept_rev_mentions_question, c1_mrel_concept_rev_mentions_question, c1_prel_concept_rev_mentions_question, x_question, x_answer, x_concept, edge_question_has_answer, edge_answer_rev_has_question, edge_question_mentions_concept, edge_concept_rev_mentions_question):
    c0 = {
        "k_w": {"question": c0_k_w_question, "answer": c0_k_w_answer, "concept": c0_k_w_concept},
        "k_b": {"question": c0_k_b_question, "answer": c0_k_b_answer, "concept": c0_k_b_concept},
        "q_w": {"question": c0_q_w_question, "answer": c0_q_w_answer, "concept": c0_q_w_concept},
        "q_b": {"question": c0_q_b_question, "answer": c0_q_b_answer, "concept": c0_q_b_concept},
        "v_w": {"question": c0_v_w_question, "answer": c0_v_w_answer, "concept": c0_v_w_concept},
        "v_b": {"question": c0_v_b_question, "answer": c0_v_b_answer, "concept": c0_v_b_concept},
        "alin_w": {"question": c0_alin_w_question, "answer": c0_alin_w_answer, "concept": c0_alin_w_concept},
        "alin_b": {"question": c0_alin_b_question, "answer": c0_alin_b_answer, "concept": c0_alin_b_concept},
        "skip": {"question": c0_skip_question, "answer": c0_skip_answer, "concept": c0_skip_concept},
        "arel": {"has": c0_arel_question_has_answer, "rev_has": c0_arel_answer_rev_has_question,
                 "mentions": c0_arel_question_mentions_concept, "rev_mentions": c0_arel_concept_rev_mentions_question},
        "mrel": {"has": c0_mrel_question_has_answer, "rev_has": c0_mrel_answer_rev_has_question,
                 "mentions": c0_mrel_question_mentions_concept, "rev_mentions": c0_mrel_concept_rev_mentions_question},
        "prel": {"has": c0_prel_question_has_answer, "rev_has": c0_prel_answer_rev_has_question,
                 "mentions": c0_prel_question_mentions_concept, "rev_mentions": c0_prel_concept_rev_mentions_question},
    }
    c1 = {
        "k_w": {"question": c1_k_w_question, "answer": c1_k_w_answer, "concept": c1_k_w_concept},
        "k_b": {"question": c1_k_b_question, "answer": c1_k_b_answer, "concept": c1_k_b_concept},
        "q_w": {"question": c1_q_w_question, "answer": c1_q_w_answer, "concept": c1_q_w_concept},
        "q_b": {"question": c1_q_b_question, "answer": c1_q_b_answer, "concept": c1_q_b_concept},
        "v_w": {"question": c1_v_w_question, "answer": c1_v_w_answer, "concept": c1_v_w_concept},
        "v_b": {"question": c1_v_b_question, "answer": c1_v_b_answer, "concept": c1_v_b_concept},
        "alin_w": {"question": c1_alin_w_question, "answer": c1_alin_w_answer, "concept": c1_alin_w_concept},
        "alin_b": {"question": c1_alin_b_question, "answer": c1_alin_b_answer, "concept": c1_alin_b_concept},
        "skip": {"question": c1_skip_question, "answer": c1_skip_answer, "concept": c1_skip_concept},
        "arel": {"has": c1_arel_question_has_answer, "rev_has": c1_arel_answer_rev_has_question,
                 "mentions": c1_arel_question_mentions_concept, "rev_mentions": c1_arel_concept_rev_mentions_question},
        "mrel": {"has": c1_mrel_question_has_answer, "rev_has": c1_mrel_answer_rev_has_question,
                 "mentions": c1_mrel_question_mentions_concept, "rev_mentions": c1_mrel_concept_rev_mentions_question},
        "prel": {"has": c1_prel_question_has_answer, "rev_has": c1_prel_answer_rev_has_question,
                 "mentions": c1_prel_question_mentions_concept, "rev_mentions": c1_prel_concept_rev_mentions_question},
    }
    et_src = {"has": "question", "rev_has": "answer",
              "mentions": "question", "rev_mentions": "concept"}

    types = ("question", "answer", "concept")
    ets = ("has", "rev_has", "mentions", "rev_mentions")

    rels, kvw, kvb = [], [], []
    for cl in (c0, c1):
        for et in ets:
            s = et_src[et]
            rels.append(cl["arel"][et]
                        * (cl["prel"][et] / math.sqrt(_HD))[:, None, None])
            kvw.append(cl["k_w"][s])
            kvb.append(cl["k_b"][s])
            rels.append(cl["mrel"][et])
            kvw.append(cl["v_w"][s])
            kvb.append(cl["v_b"][s])
    rel = jnp.stack(rels)
    w_kv = jnp.einsum("bchd,bhde->bche",
                      jnp.stack(kvw).reshape(16, _CH, _H, _HD), rel,
                      preferred_element_type=jnp.float32)
    w_kv = w_kv.reshape(16, _CH, _CH).astype(_BF16)
    b_kv = jnp.einsum("bhd,bhde->bhe",
                      jnp.stack(kvb).reshape(16, _H, _HD), rel,
                      preferred_element_type=jnp.float32).reshape(16, _CH)

    w_qa = jnp.stack([c0["q_w"][t] for t in types]
                     + [c0["alin_w"][t] for t in types]
                     + [c1["q_w"][t] for t in types]
                     + [c1["alin_w"][t] for t in types]).astype(_BF16)
    w_lin = jnp.concatenate([lin_w_question, lin_w_answer,
                             lin_w_concept], axis=0).astype(_BF16)

    alphas = jax.nn.sigmoid(jnp.stack(
        [c0["skip"][t] for t in types] + [c1["skip"][t] for t in types]))
    rows = jnp.concatenate(
        [lin_b_question, lin_b_answer, lin_b_concept,
         bn_gamma_question, bn_gamma_answer, bn_gamma_concept,
         bn_beta_question, bn_beta_answer, bn_beta_concept]
        + [c0["q_b"][t] for t in types] + [c0["alin_b"][t] for t in types]
        + [c1["q_b"][t] for t in types] + [c1["alin_b"][t] for t in types]
        + [jnp.broadcast_to(alphas[:, None], (6, _CH))], axis=0)

    ins = [x_question.astype(_BF16), x_answer.astype(_BF16),
           x_concept.astype(_BF16), w_lin, w_qa, w_kv, rows, b_kv,
           edge_question_has_answer, edge_answer_rev_has_question,
           edge_question_mentions_concept, edge_concept_rev_mentions_question]

    out = pl.pallas_call(
        _body,
        out_shape=(jax.ShapeDtypeStruct((_NQ, _CH), jnp.float32),
                   jax.ShapeDtypeStruct((_NA, _CH), jnp.float32),
                   jax.ShapeDtypeStruct((_NC, _CH), jnp.float32)),
        in_specs=[pl.BlockSpec(memory_space=pltpu.MemorySpace.VMEM)] * len(ins),
        out_specs=(pl.BlockSpec(memory_space=pltpu.MemorySpace.VMEM),) * 3,
        scratch_shapes=[pltpu.VMEM((_NTOT, _CH), _BF16),
                        pltpu.VMEM((_NTOT, _CH), _BF16),
                        pltpu.VMEM((_NQ, _NA + _NC), jnp.float32),
                        pltpu.VMEM((_NA, _NQ), jnp.float32),
                        pltpu.VMEM((_NC, _NQ), jnp.float32)],
        compiler_params=pltpu.CompilerParams(
            vmem_limit_bytes=56 * 1024 * 1024),
    )(*ins)
    return {"question": out[0], "answer": out[1], "concept": out[2]}


# speedup vs baseline: 1.1775x; 1.1775x over previous
import functools
import math

import jax
import jax.numpy as jnp
from jax.experimental import pallas as pl
from jax.experimental.pallas import tpu as pltpu

_BF16 = jnp.bfloat16
_SQRT2 = math.sqrt(2.0)

_CH = 512
_H = 8
_HD = 64
_NQ, _NA, _NC = 512, 1024, 768
_NTOT = _NQ + _NA + _NC
_ROWS = {"question": (0, 512), "answer": (512, 1536), "concept": (1536, 2304)}
_NEG = -1e30
_TYPES = ("question", "answer", "concept")
_ETS = (("has", "question", "answer"),
        ("rev_has", "answer", "question"),
        ("mentions", "question", "concept"),
        ("rev_mentions", "concept", "question"))


def _erf(x):
    a1, a2, a3, a4, a5 = 0.254829592, -0.284496736, 1.421413741, -1.453152027, 1.061405429
    p = 0.3275911
    sgn = jnp.where(x >= 0.0, 1.0, -1.0)
    ax = jnp.abs(x)
    t = 1.0 / (1.0 + p * ax)
    poly = ((((a5 * t + a4) * t + a3) * t + a2) * t + a1) * t
    return sgn * (1.0 - poly * jnp.exp(-ax * ax))


def _gelu_exact(x):
    return 0.5 * x * (1.0 + _erf(x / _SQRT2))


def _dot(a, b):
    return jnp.dot(a, b, preferred_element_type=jnp.float32)


def _dot_nt(a, b):
    return jax.lax.dot_general(a, b, (((1,), (1,)), ((), ())),
                               preferred_element_type=jnp.float32)


def _attend(kbase, vbase, srcs, nd, qbuf, lc, alin_w, alin_b, alpha, hd, write,
            sbuf, sb16, abuf):
    for h in range(_H):
        sl = slice(h * _HD, (h + 1) * _HD)
        qh = qbuf[:, sl]
        for si, (s0, s1, c0, rk, rv) in enumerate(srcs):
            ns = s1 - s0
            tb = sbuf.at[:, 0:ns]
            wb = sb16.at[:, 0:ns]
            q2 = _dot_nt(qh, rk[h]).astype(_BF16)
            tb[...] = _dot_nt(q2, kbase[s0:s1, sl]) + lc[:, c0:c0 + ns]
            rm = jnp.max(tb[...], axis=-1, keepdims=True)
            ok = rm > -1e29
            wb[...] = jnp.exp(tb[...] - rm).astype(_BF16)
            denom = jnp.sum(wb[...].astype(jnp.float32), axis=-1, keepdims=True)
            inv = jnp.where(ok, 1.0 / denom, 0.0)
            o = _dot(wb[...], vbase[s0:s1, sl])
            o = _dot(o.astype(_BF16), rv[h]) * inv
            if si == 0:
                abuf[:, sl] = o
            else:
                abuf[:, sl] = abuf[:, sl] + o
    att = abuf[...]
    g = _gelu_exact(att).astype(_BF16)
    y = _dot(g, alin_w[...]) + alin_b[...]
    a = alpha[...]
    write(a * y + (1.0 - a) * hd.astype(jnp.float32))


def _build_lc(e_ref, nd, ns, out_ref, col0):
    ne = e_ref.shape[1]
    dt = jnp.float8_e4m3fn

    def f(a_ref, b_ref):
        a_ref[...] = (jax.lax.broadcasted_iota(jnp.int32, (nd, ne), 0)
                      == e_ref[1:2, :]).astype(dt)
        b_ref[...] = (jax.lax.broadcasted_iota(jnp.int32, (ns, ne), 0)
                      == e_ref[0:1, :]).astype(dt)
        cnt = _dot_nt(a_ref[...], b_ref[...])
        out_ref[:, col0:col0 + ns] = jnp.where(cnt > 0.0, jnp.log(cnt), _NEG)

    pl.run_scoped(f, pltpu.VMEM((nd, ne), dt), pltpu.VMEM((ns, ne), dt))


def _body(xq, xa, xc, w_lin, w_all, rel, rows,
          e_has, e_rev_has, e_mentions, e_rev_mentions,
          out_q, out_a, out_c, hb0, hb1, kbase, vbase, lc_q, lc_a, lc_c):
    xs = {"question": xq, "answer": xa, "concept": xc}
    lin_w = {"question": w_lin.at[0:256], "answer": w_lin.at[256:384],
             "concept": w_lin.at[384:512]}

    edges = {"has": e_has, "rev_has": e_rev_has, "mentions": e_mentions,
             "rev_mentions": e_rev_mentions}
    lc_of = {"question": lc_q, "answer": lc_a, "concept": lc_c}
    col0 = {"has": 0, "rev_has": 0, "mentions": 0, "rev_mentions": _NA}
    _build_lc(edges["rev_has"], _NQ, _NA, lc_q, 0)
    _build_lc(edges["rev_mentions"], _NQ, _NC, lc_q, _NA)
    _build_lc(edges["has"], _NA, _NQ, lc_a, 0)
    _build_lc(edges["mentions"], _NC, _NQ, lc_c, 0)

    for i, t in enumerate(_TYPES):
        r0, r1 = _ROWS[t]
        y = _dot(xs[t][...], lin_w[t][...]) + rows[i:i + 1]
        y = jnp.maximum(y, 0.0)
        n = r1 - r0
        mean = jnp.sum(y, axis=0, keepdims=True) * (1.0 / n)
        yc = y - mean
        var = jnp.sum(yc * yc, axis=0, keepdims=True) * (1.0 / n)
        y = yc * jax.lax.rsqrt(var + 1e-5) * rows[3 + i:4 + i] + rows[6 + i:7 + i]
        hb0[r0:r1] = y.astype(_BF16)

    for L, (hb_in, wr) in enumerate(((hb0, None), (hb1, None))):
        wb = 12 * L
        rb = 9 + 15 * L
        for i, t in enumerate(_TYPES):
            r0, r1 = _ROWS[t]
            h = hb_in[r0:r1]
            kbase[r0:r1] = (_dot(h, w_all[wb + 3 + i]) + rows[rb + 3 + i:rb + 4 + i]).astype(_BF16)
            vbase[r0:r1] = (_dot(h, w_all[wb + 6 + i]) + rows[rb + 6 + i:rb + 7 + i]).astype(_BF16)
        for i, t in enumerate(_TYPES):
            d0, d1 = _ROWS[t]
            nd = d1 - d0
            hd = hb_in[d0:d1]
            srcs = []
            for j, (et, s, d) in enumerate(_ETS):
                if d != t:
                    continue
                srcs.append((_ROWS[s][0], _ROWS[s][1], col0[et],
                             rel.at[8 * L + 2 * j], rel.at[8 * L + 2 * j + 1]))
            ns_max = max(s1 - s0 for s0, s1, _, _, _ in srcs)
            if L == 0:
                def write(v, _r0=d0, _r1=d1):
                    hb1[_r0:_r1] = v.astype(_BF16)
            else:
                out = {"question": out_q, "answer": out_a, "concept": out_c}[t]

                def write(v, _o=out):
                    _o[...] = v

            def scoped(sbuf, sb16, qbuf, abuf, _i=i, _t=t, _hd=hd,
                       _srcs=srcs, _nd=nd, _write=write):
                qbuf[...] = (_dot(_hd, w_all[wb + _i])
                             + rows[rb + _i:rb + 1 + _i]).astype(_BF16)
                _attend(kbase, vbase, _srcs, _nd, qbuf, lc_of[_t],
                        w_all.at[wb + 9 + _i],
                        rows.at[rb + 9 + _i:rb + 10 + _i],
                        rows.at[rb + 12 + _i:rb + 13 + _i], _hd, _write,
                        sbuf, sb16, abuf)

            pl.run_scoped(scoped,
                          pltpu.VMEM((nd, ns_max), jnp.float32),
                          pltpu.VMEM((nd, ns_max), _BF16),
                          pltpu.VMEM((nd, _CH), _BF16),
                          pltpu.VMEM((nd, _CH), jnp.float32))


def kernel(lin_w_question, lin_b_question, bn_gamma_question, bn_beta_question, lin_w_answer, lin_b_answer, bn_gamma_answer, bn_beta_answer, lin_w_concept, lin_b_concept, bn_gamma_concept, bn_beta_concept, c0_k_w_question, c0_k_b_question, c0_q_w_question, c0_q_b_question, c0_v_w_question, c0_v_b_question, c0_alin_w_question, c0_alin_b_question, c0_skip_question, c0_k_w_answer, c0_k_b_answer, c0_q_w_answer, c0_q_b_answer, c0_v_w_answer, c0_v_b_answer, c0_alin_w_answer, c0_alin_b_answer, c0_skip_answer, c0_k_w_concept, c0_k_b_concept, c0_q_w_concept, c0_q_b_concept, c0_v_w_concept, c0_v_b_concept, c0_alin_w_concept, c0_alin_b_concept, c0_skip_concept, c0_arel_question_has_answer, c0_mrel_question_has_answer, c0_prel_question_has_answer, c0_arel_answer_rev_has_question, c0_mrel_answer_rev_has_question, c0_prel_answer_rev_has_question, c0_arel_question_mentions_concept, c0_mrel_question_mentions_concept, c0_prel_question_mentions_concept, c0_arel_concept_rev_mentions_question, c0_mrel_concept_rev_mentions_question, c0_prel_concept_rev_mentions_question, c1_k_w_question, c1_k_b_question, c1_q_w_question, c1_q_b_question, c1_v_w_question, c1_v_b_question, c1_alin_w_question, c1_alin_b_question, c1_skip_question, c1_k_w_answer, c1_k_b_answer, c1_q_w_answer, c1_q_b_answer, c1_v_w_answer, c1_v_b_answer, c1_alin_w_answer, c1_alin_b_answer, c1_skip_answer, c1_k_w_concept, c1_k_b_concept, c1_q_w_concept, c1_q_b_concept, c1_v_w_concept, c1_v_b_concept, c1_alin_w_concept, c1_alin_b_concept, c1_skip_concept, c1_arel_question_has_answer, c1_mrel_question_has_answer, c1_prel_question_has_answer, c1_arel_answer_rev_has_question, c1_mrel_answer_rev_has_question, c1_prel_answer_rev_has_question, c1_arel_question_mentions_concept, c1_mrel_question_mentions_concept, c1_prel_question_mentions_concept, c1_arel_concept_rev_mentions_question, c1_mrel_concept_rev_mentions_question, c1_prel_concept_rev_mentions_question, x_question, x_answer, x_concept, edge_question_has_answer, edge_answer_rev_has_question, edge_question_mentions_concept, edge_concept_rev_mentions_question):
    c0 = {
        "k_w": (c0_k_w_question, c0_k_w_answer, c0_k_w_concept),
        "k_b": (c0_k_b_question, c0_k_b_answer, c0_k_b_concept),
        "q_w": (c0_q_w_question, c0_q_w_answer, c0_q_w_concept),
        "q_b": (c0_q_b_question, c0_q_b_answer, c0_q_b_concept),
        "v_w": (c0_v_w_question, c0_v_w_answer, c0_v_w_concept),
        "v_b": (c0_v_b_question, c0_v_b_answer, c0_v_b_concept),
        "alin_w": (c0_alin_w_question, c0_alin_w_answer, c0_alin_w_concept),
        "alin_b": (c0_alin_b_question, c0_alin_b_answer, c0_alin_b_concept),
        "skip": (c0_skip_question, c0_skip_answer, c0_skip_concept),
        "arel": (c0_arel_question_has_answer, c0_arel_answer_rev_has_question,
                 c0_arel_question_mentions_concept, c0_arel_concept_rev_mentions_question),
        "mrel": (c0_mrel_question_has_answer, c0_mrel_answer_rev_has_question,
                 c0_mrel_question_mentions_concept, c0_mrel_concept_rev_mentions_question),
        "prel": (c0_prel_question_has_answer, c0_prel_answer_rev_has_question,
                 c0_prel_question_mentions_concept, c0_prel_concept_rev_mentions_question),
    }
    c1 = {
        "k_w": (c1_k_w_question, c1_k_w_answer, c1_k_w_concept),
        "k_b": (c1_k_b_question, c1_k_b_answer, c1_k_b_concept),
        "q_w": (c1_q_w_question, c1_q_w_answer, c1_q_w_concept),
        "q_b": (c1_q_b_question, c1_q_b_answer, c1_q_b_concept),
        "v_w": (c1_v_w_question, c1_v_w_answer, c1_v_w_concept),
        "v_b": (c1_v_b_question, c1_v_b_answer, c1_v_b_concept),
        "alin_w": (c1_alin_w_question, c1_alin_w_answer, c1_alin_w_concept),
        "alin_b": (c1_alin_b_question, c1_alin_b_answer, c1_alin_b_concept),
        "skip": (c1_skip_question, c1_skip_answer, c1_skip_concept),
        "arel": (c1_arel_question_has_answer, c1_arel_answer_rev_has_question,
                 c1_arel_question_mentions_concept, c1_arel_concept_rev_mentions_question),
        "mrel": (c1_mrel_question_has_answer, c1_mrel_answer_rev_has_question,
                 c1_mrel_question_mentions_concept, c1_mrel_concept_rev_mentions_question),
        "prel": (c1_prel_question_has_answer, c1_prel_answer_rev_has_question,
                 c1_prel_question_mentions_concept, c1_prel_concept_rev_mentions_question),
    }

    w_all = jnp.stack(
        [w for cl in (c0, c1)
         for grp in ("q_w", "k_w", "v_w", "alin_w") for w in cl[grp]]
    ).astype(_BF16)
    w_lin = jnp.concatenate([lin_w_question, lin_w_answer,
                             lin_w_concept], axis=0).astype(_BF16)

    rel = jnp.stack(
        [r for cl in (c0, c1) for j in range(4)
         for r in (cl["arel"][j] * (cl["prel"][j] / math.sqrt(_HD))[:, None, None],
                   cl["mrel"][j])]).astype(_BF16)

    alphas = jax.nn.sigmoid(jnp.stack(list(c0["skip"]) + list(c1["skip"])))
    alpha_rows = jnp.broadcast_to(alphas[:, None], (6, _CH))
    rows = jnp.concatenate(
        [lin_b_question, lin_b_answer, lin_b_concept,
         bn_gamma_question, bn_gamma_answer, bn_gamma_concept,
         bn_beta_question, bn_beta_answer, bn_beta_concept]
        + [b for b in c0["q_b"] + c0["k_b"] + c0["v_b"] + c0["alin_b"]]
        + [alpha_rows[0:3]]
        + [b for b in c1["q_b"] + c1["k_b"] + c1["v_b"] + c1["alin_b"]]
        + [alpha_rows[3:6]], axis=0)

    ins = [x_question.astype(_BF16), x_answer.astype(_BF16),
           x_concept.astype(_BF16), w_lin, w_all, rel, rows,
           edge_question_has_answer, edge_answer_rev_has_question,
           edge_question_mentions_concept, edge_concept_rev_mentions_question]

    out = pl.pallas_call(
        _body,
        out_shape=(jax.ShapeDtypeStruct((_NQ, _CH), jnp.float32),
                   jax.ShapeDtypeStruct((_NA, _CH), jnp.float32),
                   jax.ShapeDtypeStruct((_NC, _CH), jnp.float32)),
        in_specs=[pl.BlockSpec(memory_space=pltpu.MemorySpace.VMEM)] * 11,
        out_specs=(pl.BlockSpec(memory_space=pltpu.MemorySpace.VMEM),) * 3,
        scratch_shapes=[pltpu.VMEM((_NTOT, _CH), _BF16),
                        pltpu.VMEM((_NTOT, _CH), _BF16),
                        pltpu.VMEM((_NTOT, _CH), _BF16),
                        pltpu.VMEM((_NTOT, _CH), _BF16),
                        pltpu.VMEM((_NQ, _NA + _NC), jnp.float32),
                        pltpu.VMEM((_NA, _NQ), jnp.float32),
                        pltpu.VMEM((_NC, _NQ), jnp.float32)],
        compiler_params=pltpu.CompilerParams(
            vmem_limit_bytes=60 * 1024 * 1024),
    )(*ins)
    return {"question": out[0], "answer": out[1], "concept": out[2]}


# speedup vs baseline: 1.3172x; 1.1186x over previous
import functools
import math

import jax
import jax.numpy as jnp
from jax.experimental import pallas as pl
from jax.experimental.pallas import tpu as pltpu

_BF16 = jnp.bfloat16
_SQRT2 = math.sqrt(2.0)

_CH = 512
_H = 8
_HD = 64
_NQ, _NA, _NC = 512, 1024, 768
_NTOT = _NQ + _NA + _NC
_ROWS = {"question": (0, 512), "answer": (512, 1536), "concept": (1536, 2304)}
_NEG = -1e30
_TYPES = ("question", "answer", "concept")
_ETS = (("has", "question", "answer"),
        ("rev_has", "answer", "question"),
        ("mentions", "question", "concept"),
        ("rev_mentions", "concept", "question"))


def _erf(x):
    a1, a2, a3, a4, a5 = 0.254829592, -0.284496736, 1.421413741, -1.453152027, 1.061405429
    p = 0.3275911
    sgn = jnp.where(x >= 0.0, 1.0, -1.0)
    ax = jnp.abs(x)
    t = 1.0 / (1.0 + p * ax)
    poly = ((((a5 * t + a4) * t + a3) * t + a2) * t + a1) * t
    return sgn * (1.0 - poly * jnp.exp(-ax * ax))


def _gelu_exact(x):
    return 0.5 * x * (1.0 + _erf(x / _SQRT2))


def _dot(a, b):
    return jnp.dot(a, b, preferred_element_type=jnp.float32)


def _dot_nt(a, b):
    return jax.lax.dot_general(a, b, (((1,), (1,)), ((), ())),
                               preferred_element_type=jnp.float32)


def _attend(kbase, vbase, srcs, nd, qbuf, lc, alin_w, alin_b, alpha, hd, write,
            sbuf, sb16, abuf, q2b, araw, bdbuf):
    for si, (s0, s1, c0, rk, rv) in enumerate(srcs):
        ns = s1 - s0
        tb = sbuf.at[:, 0:ns]
        wb = sb16.at[:, 0:ns]
        for h in range(_H):
            bd = slice(h * _HD, (h + 1) * _HD)
            bdbuf[bd, bd] = rk[h]
        q2b[...] = _dot_nt(qbuf[...], bdbuf[...]).astype(_BF16)
        for h in range(_H):
            sl = slice(h * _HD, (h + 1) * _HD)
            tb[...] = _dot_nt(q2b[:, sl], kbase[s0:s1, sl]) + lc[:, c0:c0 + ns]
            rm = jnp.max(tb[...], axis=-1, keepdims=True)
            ok = rm > -1e29
            wb[...] = jnp.exp(tb[...] - rm).astype(_BF16)
            denom = jnp.sum(wb[...].astype(jnp.float32), axis=-1, keepdims=True)
            inv = jnp.where(ok, 1.0 / denom, 0.0)
            o = _dot(wb[...], vbase[s0:s1, sl]) * inv
            araw[:, sl] = o.astype(_BF16)
        for h in range(_H):
            bd = slice(h * _HD, (h + 1) * _HD)
            bdbuf[bd, bd] = rv[h]
        oet = _dot(araw[...], bdbuf[...])
        if si == 0:
            abuf[...] = oet
        else:
            abuf[...] = abuf[...] + oet
    att = abuf[...]
    g = _gelu_exact(att).astype(_BF16)
    y = _dot(g, alin_w[...]) + alin_b[...]
    a = alpha[...]
    write(a * y + (1.0 - a) * hd.astype(jnp.float32))


def _build_lc(e_ref, nd, ns, out_ref, col0):
    ne = e_ref.shape[1]
    dt = jnp.float8_e4m3fn

    def f(a_ref, b_ref):
        a_ref[...] = (jax.lax.broadcasted_iota(jnp.int32, (nd, ne), 0)
                      == e_ref[1:2, :]).astype(dt)
        b_ref[...] = (jax.lax.broadcasted_iota(jnp.int32, (ns, ne), 0)
                      == e_ref[0:1, :]).astype(dt)
        cnt = _dot_nt(a_ref[...], b_ref[...])
        out_ref[:, col0:col0 + ns] = jnp.where(cnt > 0.0, jnp.log(cnt), _NEG)

    pl.run_scoped(f, pltpu.VMEM((nd, ne), dt), pltpu.VMEM((ns, ne), dt))


def _body(xq, xa, xc, w_lin, w_all, rel, rows,
          e_has, e_rev_has, e_mentions, e_rev_mentions,
          out_q, out_a, out_c, hb0, hb1, kbase, vbase, bdbuf,
          lc_q, lc_a, lc_c):
    xs = {"question": xq, "answer": xa, "concept": xc}
    bdbuf[...] = jnp.zeros((_CH, _CH), _BF16)
    lin_w = {"question": w_lin.at[0:256], "answer": w_lin.at[256:384],
             "concept": w_lin.at[384:512]}

    edges = {"has": e_has, "rev_has": e_rev_has, "mentions": e_mentions,
             "rev_mentions": e_rev_mentions}
    lc_of = {"question": lc_q, "answer": lc_a, "concept": lc_c}
    col0 = {"has": 0, "rev_has": 0, "mentions": 0, "rev_mentions": _NA}
    _build_lc(edges["rev_has"], _NQ, _NA, lc_q, 0)
    _build_lc(edges["rev_mentions"], _NQ, _NC, lc_q, _NA)
    _build_lc(edges["has"], _NA, _NQ, lc_a, 0)
    _build_lc(edges["mentions"], _NC, _NQ, lc_c, 0)

    for i, t in enumerate(_TYPES):
        r0, r1 = _ROWS[t]
        y = _dot(xs[t][...], lin_w[t][...]) + rows[i:i + 1]
        y = jnp.maximum(y, 0.0)
        n = r1 - r0
        mean = jnp.sum(y, axis=0, keepdims=True) * (1.0 / n)
        yc = y - mean
        var = jnp.sum(yc * yc, axis=0, keepdims=True) * (1.0 / n)
        y = yc * jax.lax.rsqrt(var + 1e-5) * rows[3 + i:4 + i] + rows[6 + i:7 + i]
        hb0[r0:r1] = y.astype(_BF16)

    for L, (hb_in, wr) in enumerate(((hb0, None), (hb1, None))):
        wb = 12 * L
        rb = 9 + 15 * L
        for i, t in enumerate(_TYPES):
            r0, r1 = _ROWS[t]
            h = hb_in[r0:r1]
            kbase[r0:r1] = (_dot(h, w_all[wb + 3 + i]) + rows[rb + 3 + i:rb + 4 + i]).astype(_BF16)
            vbase[r0:r1] = (_dot(h, w_all[wb + 6 + i]) + rows[rb + 6 + i:rb + 7 + i]).astype(_BF16)
        for i, t in enumerate(_TYPES):
            d0, d1 = _ROWS[t]
            nd = d1 - d0
            hd = hb_in[d0:d1]
            srcs = []
            for j, (et, s, d) in enumerate(_ETS):
                if d != t:
                    continue
                srcs.append((_ROWS[s][0], _ROWS[s][1], col0[et],
                             rel.at[8 * L + 2 * j], rel.at[8 * L + 2 * j + 1]))
            ns_max = max(s1 - s0 for s0, s1, _, _, _ in srcs)
            if L == 0:
                def write(v, _r0=d0, _r1=d1):
                    hb1[_r0:_r1] = v.astype(_BF16)
            else:
                out = {"question": out_q, "answer": out_a, "concept": out_c}[t]

                def write(v, _o=out):
                    _o[...] = v

            def scoped(sbuf, sb16, qbuf, abuf, q2b, araw, _i=i, _t=t, _hd=hd,
                       _srcs=srcs, _nd=nd, _write=write):
                qbuf[...] = (_dot(_hd, w_all[wb + _i])
                             + rows[rb + _i:rb + 1 + _i]).astype(_BF16)
                _attend(kbase, vbase, _srcs, _nd, qbuf, lc_of[_t],
                        w_all.at[wb + 9 + _i],
                        rows.at[rb + 9 + _i:rb + 10 + _i],
                        rows.at[rb + 12 + _i:rb + 13 + _i], _hd, _write,
                        sbuf, sb16, abuf, q2b, araw, bdbuf)

            pl.run_scoped(scoped,
                          pltpu.VMEM((nd, ns_max), jnp.float32),
                          pltpu.VMEM((nd, ns_max), _BF16),
                          pltpu.VMEM((nd, _CH), _BF16),
                          pltpu.VMEM((nd, _CH), jnp.float32),
                          pltpu.VMEM((nd, _CH), _BF16),
                          pltpu.VMEM((nd, _CH), _BF16))


def kernel(lin_w_question, lin_b_question, bn_gamma_question, bn_beta_question, lin_w_answer, lin_b_answer, bn_gamma_answer, bn_beta_answer, lin_w_concept, lin_b_concept, bn_gamma_concept, bn_beta_concept, c0_k_w_question, c0_k_b_question, c0_q_w_question, c0_q_b_question, c0_v_w_question, c0_v_b_question, c0_alin_w_question, c0_alin_b_question, c0_skip_question, c0_k_w_answer, c0_k_b_answer, c0_q_w_answer, c0_q_b_answer, c0_v_w_answer, c0_v_b_answer, c0_alin_w_answer, c0_alin_b_answer, c0_skip_answer, c0_k_w_concept, c0_k_b_concept, c0_q_w_concept, c0_q_b_concept, c0_v_w_concept, c0_v_b_concept, c0_alin_w_concept, c0_alin_b_concept, c0_skip_concept, c0_arel_question_has_answer, c0_mrel_question_has_answer, c0_prel_question_has_answer, c0_arel_answer_rev_has_question, c0_mrel_answer_rev_has_question, c0_prel_answer_rev_has_question, c0_arel_question_mentions_concept, c0_mrel_question_mentions_concept, c0_prel_question_mentions_concept, c0_arel_concept_rev_mentions_question, c0_mrel_concept_rev_mentions_question, c0_prel_concept_rev_mentions_question, c1_k_w_question, c1_k_b_question, c1_q_w_question, c1_q_b_question, c1_v_w_question, c1_v_b_question, c1_alin_w_question, c1_alin_b_question, c1_skip_question, c1_k_w_answer, c1_k_b_answer, c1_q_w_answer, c1_q_b_answer, c1_v_w_answer, c1_v_b_answer, c1_alin_w_answer, c1_alin_b_answer, c1_skip_answer, c1_k_w_concept, c1_k_b_concept, c1_q_w_concept, c1_q_b_concept, c1_v_w_concept, c1_v_b_concept, c1_alin_w_concept, c1_alin_b_concept, c1_skip_concept, c1_arel_question_has_answer, c1_mrel_question_has_answer, c1_prel_question_has_answer, c1_arel_answer_rev_has_question, c1_mrel_answer_rev_has_question, c1_prel_answer_rev_has_question, c1_arel_question_mentions_concept, c1_mrel_question_mentions_concept, c1_prel_question_mentions_concept, c1_arel_concept_rev_mentions_question, c1_mrel_concept_rev_mentions_question, c1_prel_concept_rev_mentions_question, x_question, x_answer, x_concept, edge_question_has_answer, edge_answer_rev_has_question, edge_question_mentions_concept, edge_concept_rev_mentions_question):
    c0 = {
        "k_w": (c0_k_w_question, c0_k_w_answer, c0_k_w_concept),
        "k_b": (c0_k_b_question, c0_k_b_answer, c0_k_b_concept),
        "q_w": (c0_q_w_question, c0_q_w_answer, c0_q_w_concept),
        "q_b": (c0_q_b_question, c0_q_b_answer, c0_q_b_concept),
        "v_w": (c0_v_w_question, c0_v_w_answer, c0_v_w_concept),
        "v_b": (c0_v_b_question, c0_v_b_answer, c0_v_b_concept),
        "alin_w": (c0_alin_w_question, c0_alin_w_answer, c0_alin_w_concept),
        "alin_b": (c0_alin_b_question, c0_alin_b_answer, c0_alin_b_concept),
        "skip": (c0_skip_question, c0_skip_answer, c0_skip_concept),
        "arel": (c0_arel_question_has_answer, c0_arel_answer_rev_has_question,
                 c0_arel_question_mentions_concept, c0_arel_concept_rev_mentions_question),
        "mrel": (c0_mrel_question_has_answer, c0_mrel_answer_rev_has_question,
                 c0_mrel_question_mentions_concept, c0_mrel_concept_rev_mentions_question),
        "prel": (c0_prel_question_has_answer, c0_prel_answer_rev_has_question,
                 c0_prel_question_mentions_concept, c0_prel_concept_rev_mentions_question),
    }
    c1 = {
        "k_w": (c1_k_w_question, c1_k_w_answer, c1_k_w_concept),
        "k_b": (c1_k_b_question, c1_k_b_answer, c1_k_b_concept),
        "q_w": (c1_q_w_question, c1_q_w_answer, c1_q_w_concept),
        "q_b": (c1_q_b_question, c1_q_b_answer, c1_q_b_concept),
        "v_w": (c1_v_w_question, c1_v_w_answer, c1_v_w_concept),
        "v_b": (c1_v_b_question, c1_v_b_answer, c1_v_b_concept),
        "alin_w": (c1_alin_w_question, c1_alin_w_answer, c1_alin_w_concept),
        "alin_b": (c1_alin_b_question, c1_alin_b_answer, c1_alin_b_concept),
        "skip": (c1_skip_question, c1_skip_answer, c1_skip_concept),
        "arel": (c1_arel_question_has_answer, c1_arel_answer_rev_has_question,
                 c1_arel_question_mentions_concept, c1_arel_concept_rev_mentions_question),
        "mrel": (c1_mrel_question_has_answer, c1_mrel_answer_rev_has_question,
                 c1_mrel_question_mentions_concept, c1_mrel_concept_rev_mentions_question),
        "prel": (c1_prel_question_has_answer, c1_prel_answer_rev_has_question,
                 c1_prel_question_mentions_concept, c1_prel_concept_rev_mentions_question),
    }

    w_all = jnp.stack(
        [w for cl in (c0, c1)
         for grp in ("q_w", "k_w", "v_w", "alin_w") for w in cl[grp]]
    ).astype(_BF16)
    w_lin = jnp.concatenate([lin_w_question, lin_w_answer,
                             lin_w_concept], axis=0).astype(_BF16)

    rel = jnp.stack(
        [r for cl in (c0, c1) for j in range(4)
         for r in (cl["arel"][j] * (cl["prel"][j] / math.sqrt(_HD))[:, None, None],
                   cl["mrel"][j])]).astype(_BF16)

    alphas = jax.nn.sigmoid(jnp.stack(list(c0["skip"]) + list(c1["skip"])))
    alpha_rows = jnp.broadcast_to(alphas[:, None], (6, _CH))
    rows = jnp.concatenate(
        [lin_b_question, lin_b_answer, lin_b_concept,
         bn_gamma_question, bn_gamma_answer, bn_gamma_concept,
         bn_beta_question, bn_beta_answer, bn_beta_concept]
        + [b for b in c0["q_b"] + c0["k_b"] + c0["v_b"] + c0["alin_b"]]
        + [alpha_rows[0:3]]
        + [b for b in c1["q_b"] + c1["k_b"] + c1["v_b"] + c1["alin_b"]]
        + [alpha_rows[3:6]], axis=0)

    ins = [x_question.astype(_BF16), x_answer.astype(_BF16),
           x_concept.astype(_BF16), w_lin, w_all, rel, rows,
           edge_question_has_answer, edge_answer_rev_has_question,
           edge_question_mentions_concept, edge_concept_rev_mentions_question]

    out = pl.pallas_call(
        _body,
        out_shape=(jax.ShapeDtypeStruct((_NQ, _CH), jnp.float32),
                   jax.ShapeDtypeStruct((_NA, _CH), jnp.float32),
                   jax.ShapeDtypeStruct((_NC, _CH), jnp.float32)),
        in_specs=[pl.BlockSpec(memory_space=pltpu.MemorySpace.VMEM)] * 11,
        out_specs=(pl.BlockSpec(memory_space=pltpu.MemorySpace.VMEM),) * 3,
        scratch_shapes=[pltpu.VMEM((_NTOT, _CH), _BF16),
                        pltpu.VMEM((_NTOT, _CH), _BF16),
                        pltpu.VMEM((_NTOT, _CH), _BF16),
                        pltpu.VMEM((_NTOT, _CH), _BF16),
                        pltpu.VMEM((_CH, _CH), _BF16),
                        pltpu.VMEM((_NQ, _NA + _NC), jnp.float32),
                        pltpu.VMEM((_NA, _NQ), jnp.float32),
                        pltpu.VMEM((_NC, _NQ), jnp.float32)],
        compiler_params=pltpu.CompilerParams(
            vmem_limit_bytes=60 * 1024 * 1024),
    )(*ins)
    return {"question": out[0], "answer": out[1], "concept": out[2]}
